# baseline (speedup 1.0000x reference)
import jax
import jax.numpy as jnp
import numpy as np
from jax.experimental import pallas as pl
from jax.experimental.pallas import tpu as pltpu

_N = 8192
_B = 512
_T = _N // _B
_NBLK = _T * (_T + 1) // 2

_RMAP, _CMAP = (np.array(x, dtype=np.int32) for x in zip(
    *[(r, c) for r in range(_T) for c in range(r, _T)]))

_NEG_LOG2E = -1.4426950408889634
_TWO_LN2 = 1.3862943611198906
_LN2 = 0.6931471805599453


def _body(rmap, cmap, pr, tr, pc, tc, out, acc):
    i = pl.program_id(0)
    r = rmap[i]
    c = cmap[i]

    @pl.when(i == 0)
    def _init():
        acc[...] = jnp.zeros_like(acc)

    pc_v = pc[...]
    tc_v = tc[...]

    zeros = jnp.zeros((8, _B), jnp.float32)
    m_acc = [zeros, zeros]
    g_acc = [zeros, zeros]
    for k in range(_B // 8):
        pr_s = pr[8 * k:8 * k + 8, :]
        tr_s = tr[8 * k:8 * k + 8, :]
        d = pr_s - pc_v
        a = jnp.abs(d)
        g = jnp.log2(1.0 + jnp.exp2(a * _NEG_LOG2E))
        x = (a
             - jnp.where(tr_s > tc_v, d, 0.0)
             + jnp.where(tr_s < tc_v, d, 0.0))
        m_acc[k % 2] = m_acc[k % 2] + x
        g_acc[k % 2] = g_acc[k % 2] + g
    w = jnp.where(r == c, 0.5, 1.0)
    acc[0:8, :] += w * (m_acc[0] + m_acc[1])
    acc[8:16, :] += w * (g_acc[0] + g_acc[1])

    @pl.when(i == _NBLK - 1)
    def _fold():
        def fold8(a8):
            tot = a8[:, 0:128]
            for l in range(1, _B // 128):
                tot = tot + a8[:, 128 * l:128 * (l + 1)]
            return tot

        s = fold8(acc[0:8, :]) + _TWO_LN2 * fold8(acc[8:16, :])
        total = jnp.sum(s)
        out[0, 0, 0] = (total - _N * _LN2) / (_N * (_N - 1.0))


def _loss(p_row, t_row, p_col, t_col):
    return pl.pallas_call(
        _body,
        grid_spec=pltpu.PrefetchScalarGridSpec(
            num_scalar_prefetch=2,
            grid=(_NBLK,),
            in_specs=[
                pl.BlockSpec((_B, 1), lambda i, rm, cm: (rm[i], 0)),
                pl.BlockSpec((_B, 1), lambda i, rm, cm: (rm[i], 0)),
                pl.BlockSpec((1, _B), lambda i, rm, cm: (0, cm[i])),
                pl.BlockSpec((1, _B), lambda i, rm, cm: (0, cm[i])),
            ],
            out_specs=pl.BlockSpec((1, 1, 1), lambda i, rm, cm: (0, 0, 0),
                                   memory_space=pltpu.SMEM),
            scratch_shapes=[pltpu.VMEM((16, _B), jnp.float32)],
        ),
        out_shape=jax.ShapeDtypeStruct((1, 1, 1), jnp.float32),
        compiler_params=pltpu.CompilerParams(
            dimension_semantics=("arbitrary",),
        ),
    )(jnp.asarray(_RMAP), jnp.asarray(_CMAP), p_row, t_row, p_col, t_col)


def kernel(preds, targets):
    n = preds.shape[0]
    p_row = preds.reshape(n, 1)
    t_row = targets.reshape(n, 1)
    p_col = preds.reshape(1, n)
    t_col = targets.reshape(1, n)
    res = _loss(p_row, t_row, p_col, t_col)
    return res.reshape(())


# speedup vs baseline: 1.1668x; 1.1668x over previous
import jax
import jax.numpy as jnp
import numpy as np
from jax.experimental import pallas as pl
from jax.experimental.pallas import tpu as pltpu

_N = 8192
_B = 1024
_T = _N // _B
_NBLK = _T * (_T + 1) // 2

_RMAP, _CMAP = (np.array(x, dtype=np.int32) for x in zip(
    *[(r, c) for r in range(_T) for c in range(r, _T)]))

_NEG_LOG2E = -1.4426950408889634
_TWO_LN2 = 1.3862943611198906
_LN2 = 0.6931471805599453


def _body(rmap, cmap, pr, tr, pc, tc, out, acc):
    i = pl.program_id(0)
    r = rmap[i]
    c = cmap[i]

    @pl.when(i == 0)
    def _init():
        acc[...] = jnp.zeros_like(acc)

    pc_v = pc[...]
    tc_v = tc[...]

    zeros = jnp.zeros((8, _B), jnp.float32)
    m_acc = [zeros, zeros]
    g_acc = [zeros, zeros]
    for k in range(_B // 8):
        pr_s = pr[8 * k:8 * k + 8, :]
        tr_s = tr[8 * k:8 * k + 8, :]
        d = pr_s - pc_v
        a = jnp.abs(d)
        g = jnp.log2(1.0 + jnp.exp2(a * _NEG_LOG2E))
        x = (a
             - jnp.where(tr_s > tc_v, d, 0.0)
             + jnp.where(tr_s < tc_v, d, 0.0))
        m_acc[k % 2] = m_acc[k % 2] + x
        g_acc[k % 2] = g_acc[k % 2] + g
    w = jnp.where(r == c, 0.5, 1.0)
    acc[0:8, :] += w * (m_acc[0] + m_acc[1])
    acc[8:16, :] += w * (g_acc[0] + g_acc[1])

    @pl.when(i == _NBLK - 1)
    def _fold():
        def fold8(a8):
            tot = a8[:, 0:128]
            for l in range(1, _B // 128):
                tot = tot + a8[:, 128 * l:128 * (l + 1)]
            return tot

        s = fold8(acc[0:8, :]) + _TWO_LN2 * fold8(acc[8:16, :])
        total = jnp.sum(s)
        out[0, 0, 0] = (total - _N * _LN2) / (_N * (_N - 1.0))


def _loss(p_row, t_row, p_col, t_col):
    return pl.pallas_call(
        _body,
        grid_spec=pltpu.PrefetchScalarGridSpec(
            num_scalar_prefetch=2,
            grid=(_NBLK,),
            in_specs=[
                pl.BlockSpec((_B, 1), lambda i, rm, cm: (rm[i], 0)),
                pl.BlockSpec((_B, 1), lambda i, rm, cm: (rm[i], 0)),
                pl.BlockSpec((1, _B), lambda i, rm, cm: (0, cm[i])),
                pl.BlockSpec((1, _B), lambda i, rm, cm: (0, cm[i])),
            ],
            out_specs=pl.BlockSpec((1, 1, 1), lambda i, rm, cm: (0, 0, 0),
                                   memory_space=pltpu.SMEM),
            scratch_shapes=[pltpu.VMEM((16, _B), jnp.float32)],
        ),
        out_shape=jax.ShapeDtypeStruct((1, 1, 1), jnp.float32),
        compiler_params=pltpu.CompilerParams(
            dimension_semantics=("arbitrary",),
        ),
    )(jnp.asarray(_RMAP), jnp.asarray(_CMAP), p_row, t_row, p_col, t_col)


def kernel(preds, targets):
    n = preds.shape[0]
    p_row = preds.reshape(n, 1)
    t_row = targets.reshape(n, 1)
    p_col = preds.reshape(1, n)
    t_col = targets.reshape(1, n)
    res = _loss(p_row, t_row, p_col, t_col)
    return res.reshape(())


# speedup vs baseline: 1.1763x; 1.0081x over previous
import jax
import jax.numpy as jnp
import numpy as np
from jax.experimental import pallas as pl
from jax.experimental.pallas import tpu as pltpu

_N = 8192
_B = 1024
_T = _N // _B
_NBLK = _T * (_T + 1) // 2

_RMAP, _CMAP = (np.array(x, dtype=np.int32) for x in zip(
    *[(r, c) for r in range(_T) for c in range(r, _T)]))

_NEG_LOG2E = -1.4426950408889634
_TWO_LN2 = 1.3862943611198906
_LN2 = 0.6931471805599453


def _body(rmap, cmap, pr, tr, pc, tc, out, acc):
    i = pl.program_id(0)
    r = rmap[i]
    c = cmap[i]

    @pl.when(i == 0)
    def _init():
        acc[...] = jnp.zeros_like(acc)

    pc_v = pc[...]
    tc_v = tc[...]

    zeros = jnp.zeros((8, _B), jnp.float32)
    m_acc = [zeros, zeros, zeros, zeros]
    g_acc = [zeros, zeros, zeros, zeros]
    for k in range(_B // 8):
        pr_s = pr[8 * k:8 * k + 8, :]
        tr_s = tr[8 * k:8 * k + 8, :]
        d = pr_s - pc_v
        a = jnp.abs(d)
        g = jnp.log2(1.0 + jnp.exp2(a * _NEG_LOG2E))
        x = (a
             - jnp.where(tr_s > tc_v, d, 0.0)
             + jnp.where(tr_s < tc_v, d, 0.0))
        m_acc[k % 4] = m_acc[k % 4] + x
        g_acc[k % 4] = g_acc[k % 4] + g
    w = jnp.where(r == c, 0.5, 1.0)
    acc[0:8, :] += w * ((m_acc[0] + m_acc[1]) + (m_acc[2] + m_acc[3]))
    acc[8:16, :] += w * ((g_acc[0] + g_acc[1]) + (g_acc[2] + g_acc[3]))

    @pl.when(i == _NBLK - 1)
    def _fold():
        def fold8(a8):
            tot = a8[:, 0:128]
            for l in range(1, _B // 128):
                tot = tot + a8[:, 128 * l:128 * (l + 1)]
            return tot

        s = fold8(acc[0:8, :]) + _TWO_LN2 * fold8(acc[8:16, :])
        total = jnp.sum(s)
        out[0, 0, 0] = (total - _N * _LN2) / (_N * (_N - 1.0))


def _loss(p_row, t_row, p_col, t_col):
    return pl.pallas_call(
        _body,
        grid_spec=pltpu.PrefetchScalarGridSpec(
            num_scalar_prefetch=2,
            grid=(_NBLK,),
            in_specs=[
                pl.BlockSpec((_B, 1), lambda i, rm, cm: (rm[i], 0)),
                pl.BlockSpec((_B, 1), lambda i, rm, cm: (rm[i], 0)),
                pl.BlockSpec((1, _B), lambda i, rm, cm: (0, cm[i])),
                pl.BlockSpec((1, _B), lambda i, rm, cm: (0, cm[i])),
            ],
            out_specs=pl.BlockSpec((1, 1, 1), lambda i, rm, cm: (0, 0, 0),
                                   memory_space=pltpu.SMEM),
            scratch_shapes=[pltpu.VMEM((16, _B), jnp.float32)],
        ),
        out_shape=jax.ShapeDtypeStruct((1, 1, 1), jnp.float32),
        compiler_params=pltpu.CompilerParams(
            dimension_semantics=("arbitrary",),
        ),
    )(jnp.asarray(_RMAP), jnp.asarray(_CMAP), p_row, t_row, p_col, t_col)


def kernel(preds, targets):
    n = preds.shape[0]
    p_row = preds.reshape(n, 1)
    t_row = targets.reshape(n, 1)
    p_col = preds.reshape(1, n)
    t_col = targets.reshape(1, n)
    res = _loss(p_row, t_row, p_col, t_col)
    return res.reshape(())
